# baseline (speedup 1.0000x reference)
import functools

import jax
import jax.numpy as jnp
from jax import lax
from jax.experimental import pallas as pl
from jax.experimental.pallas import tpu as pltpu
from jax.experimental.pallas import tpu_sc as plsc

VOCAB = 33
HIDDEN = 512
NUM_TABLES = 6
B = 4 * 8192
NC, NS = 2, 16
NW = NC * NS
L = 16
K = HIDDEN // L
ROWS_PER_W = B // NW
CHUNK = 16
NCHUNK = ROWS_PER_W // CHUNK
NBUF = 3


def _make_sc_lookup():
  mesh = plsc.VectorSubcoreMesh(
      core_axis_name="c", subcore_axis_name="s", num_cores=NC, num_subcores=NS
  )
  out_type = [
      jax.ShapeDtypeStruct((B, HIDDEN), jnp.float32) for _ in range(NUM_TABLES)
  ]
  scratch = [
      pltpu.VMEM((ROWS_PER_W // 128, 128), jnp.int32),
      [pltpu.VMEM((VOCAB * HIDDEN,), jnp.float32) for _ in range(NUM_TABLES)],
      [pltpu.VMEM((CHUNK, HIDDEN), jnp.float32) for _ in range(NBUF)],
      pltpu.SemaphoreType.DMA,
  ]

  @functools.partial(
      pl.kernel, mesh=mesh, out_type=out_type, scratch_types=scratch,
      compiler_params=pltpu.CompilerParams(needs_layout_passes=False),
  )
  def lookup(idx_hbm, t0, t1, t2, t3, t4, t5, o0, o1, o2, o3, o4, o5,
             idx_v, tbls, bufs, ssem):
    outs = (o0, o1, o2, o3, o4, o5)
    wid = lax.axis_index("s") * NC + lax.axis_index("c")
    base0 = wid * ROWS_PER_W

    pltpu.sync_copy(idx_hbm.at[wid], idx_v)
    for src, dst in zip((t0, t1, t2, t3, t4, t5), tbls):
      pltpu.sync_copy(src, dst)

    cols = [jnp.arange(L, dtype=jnp.int32) + L * k for k in range(K)]

    def wait_store():
      pltpu.make_async_copy(bufs[0], o0.at[pl.ds(0, CHUNK)], ssem).wait()

    def chunk_body(ci, carry):
      irow = jnp.full((L,), ci // 8, jnp.int32)
      icol0 = (ci % 8) * L
      for t in range(NUM_TABLES):
        if t < NBUF:
          @pl.when(ci > 0)
          def _():
            wait_store()
        else:
          wait_store()

        buf = bufs[t % NBUF]

        def fill_row(j, cy, t=t, buf=buf, irow=irow, icol0=icol0):
          bc = plsc.load_gather(idx_v, [irow, jnp.full((L,), icol0 + j,
                                                       jnp.int32)])
          flat0 = bc * HIDDEN
          for k in range(K):
            buf[j, pl.ds(L * k, L)] = plsc.load_gather(
                tbls[t], [flat0 + cols[k]])
          return cy

        lax.fori_loop(0, CHUNK, fill_row, 0)
        pltpu.async_copy(buf, outs[t].at[pl.ds(base0 + ci * CHUNK, CHUNK)],
                         ssem)
      return carry

    lax.fori_loop(0, NCHUNK, chunk_body, 0)
    for _ in range(NBUF):
      wait_store()

  return lookup


_sc_lookup = _make_sc_lookup()


def kernel(inputs, tables):
  idx = inputs.reshape(NW, ROWS_PER_W // 128, 128).astype(jnp.int32)
  tbls = [tables[i].reshape(-1) for i in range(NUM_TABLES)]
  flat = _sc_lookup(idx, *tbls)
  ve = [o.reshape(inputs.shape + (HIDDEN,)) for o in flat]
  return tuple(ve + list(reversed(ve)))


# speedup vs baseline: 2.0989x; 2.0989x over previous
import functools

import jax
import jax.numpy as jnp
from jax import lax
from jax.experimental import pallas as pl
from jax.experimental.pallas import tpu as pltpu
from jax.experimental.pallas import tpu_sc as plsc

VOCAB = 33
HIDDEN = 512
NUM_TABLES = 6
B = 4 * 8192
NC, NS = 2, 16
NW = NC * NS
L = 16
K = HIDDEN // L
ROWS_PER_W = B // NW
CHUNK = 16
NCHUNK = ROWS_PER_W // CHUNK
NBUF = 3


def _make_sc_lookup():
  mesh = plsc.VectorSubcoreMesh(
      core_axis_name="c", subcore_axis_name="s", num_cores=NC, num_subcores=NS
  )
  out_type = [
      jax.ShapeDtypeStruct((B, HIDDEN), jnp.float32) for _ in range(NUM_TABLES)
  ]
  scratch = [
      pltpu.VMEM((ROWS_PER_W // 128, 128), jnp.int32),
      [pltpu.VMEM((VOCAB * HIDDEN,), jnp.float32) for _ in range(NUM_TABLES)],
      [pltpu.VMEM((CHUNK, HIDDEN), jnp.float32) for _ in range(NBUF)],
      pltpu.SemaphoreType.DMA,
  ]

  @functools.partial(
      pl.kernel, mesh=mesh, out_type=out_type, scratch_types=scratch,
      compiler_params=pltpu.CompilerParams(needs_layout_passes=False),
  )
  def lookup(idx_hbm, t0, t1, t2, t3, t4, t5, o0, o1, o2, o3, o4, o5,
             idx_v, tbls, bufs, ssem):
    outs = (o0, o1, o2, o3, o4, o5)
    wid = lax.axis_index("s") * NC + lax.axis_index("c")
    base0 = wid * ROWS_PER_W

    pltpu.sync_copy(idx_hbm.at[wid], idx_v)
    for src, dst in zip((t0, t1, t2, t3, t4, t5), tbls):
      pltpu.sync_copy(src, dst)

    cols = [jnp.arange(L, dtype=jnp.int32) + L * k for k in range(K)]

    def wait_store():
      pltpu.make_async_copy(bufs[0], o0.at[pl.ds(0, CHUNK)], ssem).wait()

    def chunk_body(ci, carry):
      irow = jnp.full((L,), ci // 8, jnp.int32)
      icol0 = (ci % 8) * L
      for t in range(NUM_TABLES):
        if t < NBUF:
          @pl.when(ci > 0)
          def _():
            wait_store()
        else:
          wait_store()

        buf = bufs[t % NBUF]

        @plsc.parallel_loop(0, CHUNK, 1, unroll=2)
        def _(j, t=t, buf=buf, irow=irow, icol0=icol0):
          bc = plsc.load_gather(idx_v, [irow, jnp.full((L,), icol0 + j,
                                                       jnp.int32)])
          flat0 = bc * HIDDEN
          for k in range(K):
            buf[j, pl.ds(L * k, L)] = plsc.load_gather(
                tbls[t], [flat0 + cols[k]])
        pltpu.async_copy(buf, outs[t].at[pl.ds(base0 + ci * CHUNK, CHUNK)],
                         ssem)
      return carry

    lax.fori_loop(0, NCHUNK, chunk_body, 0)
    for _ in range(NBUF):
      wait_store()

  return lookup


_sc_lookup = _make_sc_lookup()


def kernel(inputs, tables):
  idx = inputs.reshape(NW, ROWS_PER_W // 128, 128).astype(jnp.int32)
  tbls = [tables[i].reshape(-1) for i in range(NUM_TABLES)]
  flat = _sc_lookup(idx, *tbls)
  ve = [o.reshape(inputs.shape + (HIDDEN,)) for o in flat]
  return tuple(ve + list(reversed(ve)))
